# baseline (speedup 1.0000x reference)
import functools

import jax
import jax.numpy as jnp
from jax import lax
from jax.experimental import pallas as pl
from jax.experimental.pallas import tpu as pltpu
from jax.experimental.pallas import tpu_sc as plsc

_NUM_CORES = 2
_NUM_SUBCORES = 16
_NUM_WORKERS = _NUM_CORES * _NUM_SUBCORES
_LANES = 16


def _make_kernel(B, S, D, V):
    assert B % (_NUM_WORKERS * 128) == 0 and D == 32 and V % 4 == 0
    slab = B // _NUM_WORKERS
    n_grp = slab // _LANES

    mesh = plsc.VectorSubcoreMesh(core_axis_name="c", subcore_axis_name="s")

    @functools.partial(
        pl.kernel,
        mesh=mesh,
        out_type=jax.ShapeDtypeStruct((S, D, B), jnp.float32),
        scratch_types=[
            pltpu.VMEM((S, D), jnp.float32),
            pltpu.VMEM((slab,), jnp.int32),
            pltpu.VMEM((slab,), jnp.int32),
            pltpu.VMEM((slab,), jnp.int32),
            pltpu.VMEM((slab,), jnp.int32),
            pltpu.VMEM((slab,), jnp.int32),
            pltpu.VMEM((slab,), jnp.int32),
            pltpu.VMEM((slab, 128), jnp.float32),
            pltpu.VMEM((slab, 128), jnp.float32),
            pltpu.VMEM((D, slab), jnp.float32),
            pltpu.VMEM((D, slab), jnp.float32),
            pltpu.SemaphoreType.DMA,
            pltpu.SemaphoreType.DMA,
            pltpu.SemaphoreType.DMA,
            pltpu.SemaphoreType.DMA,
            pltpu.SemaphoreType.DMA,
            pltpu.SemaphoreType.DMA,
        ],
        compiler_params=pltpu.CompilerParams(use_tc_tiling_on_sc=True,
                                             needs_layout_passes=False),
    )
    def embed(ids_hbm, table_hbm, pos_hbm, out_hbm,
              pos_v, idx0, idx1, ridx0, ridx1, sub0, sub1,
              rows0, rows1, outT0, outT1,
              isem0, isem1, gsem0, gsem1, osem0, osem1):
        wid = lax.axis_index("s") * _NUM_CORES + lax.axis_index("c")
        b0 = wid * slab
        idx = (idx0, idx1)
        ridx = (ridx0, ridx1)
        sub = (sub0, sub1)
        rows = (rows0, rows1)
        outT = (outT0, outT1)
        isem = (isem0, isem1)
        gsem = (gsem0, gsem1)
        osem = (osem0, osem1)

        pltpu.sync_copy(pos_hbm, pos_v)

        def ids_src(s):
            return ids_hbm.at[s, pl.ds(b0, slab)]

        def out_dst(s):
            return out_hbm.at[s, :, pl.ds(b0, slab)]

        def split_ids(b):
            for j in range(n_grp):
                sl = pl.ds(j * _LANES, _LANES)
                v = idx[b][sl]
                ridx[b][sl] = lax.shift_right_logical(v, 2)
                sub[b][sl] = lax.shift_left(lax.bitwise_and(v, 3), 5)

        pltpu.async_copy(ids_src(0), idx[0], isem[0])
        pltpu.async_copy(ids_src(1), idx[1], isem[1])
        pltpu.make_async_copy(ids_src(0), idx[0], isem[0]).wait()
        split_ids(0)
        pltpu.async_copy(table_hbm.at[ridx[0]], rows[0], gsem[0])

        lanes = jnp.arange(_LANES, dtype=jnp.int32)

        @pl.loop(0, S, step=2)
        def chunk_loop(s0):
            for b in range(2):
                s = s0 + b
                nb = 1 - b

                @pl.when(s + 1 < S)
                def _():
                    pltpu.make_async_copy(ids_src(s + 1), idx[nb],
                                          isem[nb]).wait()
                    split_ids(nb)

                    @pl.when(s >= 1)
                    def _():
                        pltpu.make_async_copy(outT[nb], out_dst(s - 1),
                                              osem[nb]).wait()

                    pltpu.async_copy(table_hbm.at[ridx[nb]], rows[nb],
                                     gsem[nb])

                @pl.when(s + 2 < S)
                def _():
                    pltpu.async_copy(ids_src(s + 2), idx[b], isem[b])

                pltpu.make_async_copy(table_hbm.at[ridx[b]], rows[b],
                                      gsem[b]).wait()

                prow = [pos_v[s, pl.ds(h * _LANES, _LANES)]
                        for h in range(D // _LANES)]
                def splat(h, i):
                    idxv = jnp.full((_LANES,), i, jnp.int32)
                    return lax.gather(
                        prow[h], idxv[:, None],
                        lax.GatherDimensionNumbers(
                            offset_dims=(), collapsed_slice_dims=(0,),
                            start_index_map=(0,)),
                        (1,), mode=lax.GatherScatterMode.PROMISE_IN_BOUNDS)

                pspl = [splat(d // _LANES, d % _LANES) for d in range(D)]
                for j in range(n_grp):
                    sl = pl.ds(j * _LANES, _LANES)
                    tok = lanes + (j * _LANES)
                    colb = sub[b][sl]
                    for d in range(D):
                        val = plsc.load_gather(rows[b], [tok, colb + d])
                        outT[b][d, sl] = val + pspl[d]

                pltpu.async_copy(outT[b], out_dst(s), osem[b])

        pltpu.make_async_copy(outT[0], out_dst(S - 2), osem[0]).wait()
        pltpu.make_async_copy(outT[1], out_dst(S - 1), osem[1]).wait()

    return embed


def kernel(input_ids, token_table, pos_table):
    B, S = input_ids.shape
    V, D = token_table.shape
    ids_t = input_ids.astype(jnp.int32).T
    tbl = token_table.reshape(V // 4, 4 * D)
    out = _make_kernel(B, S, D, V)(ids_t, tbl, pos_table)
    return out.transpose(2, 0, 1)


# speedup vs baseline: 1.1841x; 1.1841x over previous
import functools

import jax
import jax.numpy as jnp
from jax import lax
from jax.experimental import pallas as pl
from jax.experimental.pallas import tpu as pltpu
from jax.experimental.pallas import tpu_sc as plsc

_NUM_CORES = 2
_NUM_SUBCORES = 16
_NUM_WORKERS = _NUM_CORES * _NUM_SUBCORES
_LANES = 16


def _make_kernel(B, S, D, V):
    assert B % (_NUM_WORKERS * 128) == 0 and D == 32 and V % 4 == 0
    slab = B // _NUM_WORKERS
    n_grp = slab // _LANES

    mesh = plsc.VectorSubcoreMesh(core_axis_name="c", subcore_axis_name="s")

    @functools.partial(
        pl.kernel,
        mesh=mesh,
        out_type=jax.ShapeDtypeStruct((S, D, B), jnp.float32),
        scratch_types=[
            pltpu.VMEM((S, D), jnp.float32),
            pltpu.VMEM((slab,), jnp.int32),
            pltpu.VMEM((slab,), jnp.int32),
            pltpu.VMEM((slab,), jnp.int32),
            pltpu.VMEM((slab,), jnp.int32),
            pltpu.VMEM((slab,), jnp.int32),
            pltpu.VMEM((slab,), jnp.int32),
            pltpu.VMEM((slab, 128), jnp.float32),
            pltpu.VMEM((slab, 128), jnp.float32),
            pltpu.VMEM((D, slab), jnp.float32),
            pltpu.VMEM((D, slab), jnp.float32),
            pltpu.SemaphoreType.DMA,
            pltpu.SemaphoreType.DMA,
            pltpu.SemaphoreType.DMA,
            pltpu.SemaphoreType.DMA,
            pltpu.SemaphoreType.DMA,
            pltpu.SemaphoreType.DMA,
        ],
        compiler_params=pltpu.CompilerParams(use_tc_tiling_on_sc=True,
                                             needs_layout_passes=False),
    )
    def embed(ids_hbm, table_hbm, pos_hbm, out_hbm,
              pos_v, idx0, idx1, ridx0, ridx1, sub0, sub1,
              rows0, rows1, outT0, outT1,
              isem0, isem1, gsem0, gsem1, osem0, osem1):
        wid = lax.axis_index("s") * _NUM_CORES + lax.axis_index("c")
        b0 = wid * slab
        idx = (idx0, idx1)
        ridx = (ridx0, ridx1)
        sub = (sub0, sub1)
        rows = (rows0, rows1)
        outT = (outT0, outT1)
        isem = (isem0, isem1)
        gsem = (gsem0, gsem1)
        osem = (osem0, osem1)

        pltpu.sync_copy(pos_hbm, pos_v)

        def ids_src(s):
            return ids_hbm.at[s, pl.ds(b0, slab)]

        def out_dst(s):
            return out_hbm.at[s, :, pl.ds(b0, slab)]

        def split_ids(b):
            for j in range(n_grp):
                sl = pl.ds(j * _LANES, _LANES)
                v = idx[b][sl]
                ridx[b][sl] = lax.shift_right_logical(v, 2)
                sub[b][sl] = lax.shift_left(lax.bitwise_and(v, 3), 5)

        pltpu.async_copy(ids_src(0), idx[0], isem[0])
        pltpu.async_copy(ids_src(1), idx[1], isem[1])
        pltpu.make_async_copy(ids_src(0), idx[0], isem[0]).wait()
        split_ids(0)
        pltpu.async_copy(table_hbm.at[ridx[0]], rows[0], gsem[0])

        lanes = jnp.arange(_LANES, dtype=jnp.int32)

        @pl.loop(0, S, step=2)
        def chunk_loop(s0):
            for b in range(2):
                s = s0 + b
                nb = 1 - b

                @pl.when(s + 1 < S)
                def _():
                    pltpu.make_async_copy(ids_src(s + 1), idx[nb],
                                          isem[nb]).wait()
                    split_ids(nb)

                    @pl.when(s >= 1)
                    def _():
                        pltpu.make_async_copy(outT[nb], out_dst(s - 1),
                                              osem[nb]).wait()

                    pltpu.async_copy(table_hbm.at[ridx[nb]], rows[nb],
                                     gsem[nb])

                @pl.when(s + 2 < S)
                def _():
                    pltpu.async_copy(ids_src(s + 2), idx[b], isem[b])

                pltpu.make_async_copy(table_hbm.at[ridx[b]], rows[b],
                                      gsem[b]).wait()

                prow = [pos_v[s, pl.ds(h * _LANES, _LANES)]
                        for h in range(D // _LANES)]
                def splat(h, i):
                    idxv = jnp.full((_LANES,), i, jnp.int32)
                    return lax.gather(
                        prow[h], idxv[:, None],
                        lax.GatherDimensionNumbers(
                            offset_dims=(), collapsed_slice_dims=(0,),
                            start_index_map=(0,)),
                        (1,), mode=lax.GatherScatterMode.PROMISE_IN_BOUNDS)

                toks = [lanes + (j * _LANES) for j in range(n_grp)]
                colbs = [sub[b][pl.ds(j * _LANES, _LANES)]
                         for j in range(n_grp)]
                for d in range(D):
                    pv = splat(d // _LANES, d % _LANES)
                    vals = [plsc.load_gather(rows[b], [toks[j], colbs[j] + d])
                            for j in range(n_grp)]
                    for j in range(n_grp):
                        outT[b][d, pl.ds(j * _LANES, _LANES)] = vals[j] + pv

                pltpu.async_copy(outT[b], out_dst(s), osem[b])

        pltpu.make_async_copy(outT[0], out_dst(S - 2), osem[0]).wait()
        pltpu.make_async_copy(outT[1], out_dst(S - 1), osem[1]).wait()

    return embed


def kernel(input_ids, token_table, pos_table):
    B, S = input_ids.shape
    V, D = token_table.shape
    ids_t = input_ids.astype(jnp.int32).T
    tbl = token_table.reshape(V // 4, 4 * D)
    out = _make_kernel(B, S, D, V)(ids_t, tbl, pos_table)
    return out.transpose(2, 0, 1)


# speedup vs baseline: 1.7481x; 1.4763x over previous
import functools

import jax
import jax.numpy as jnp
from jax import lax
from jax.experimental import pallas as pl
from jax.experimental.pallas import tpu as pltpu
from jax.experimental.pallas import tpu_sc as plsc

_NUM_CORES = 2
_NUM_SUBCORES = 16
_NUM_WORKERS = _NUM_CORES * _NUM_SUBCORES
_LANES = 16


def _make_kernel(B, S, D, V):
    assert B % (_NUM_WORKERS * 128) == 0 and D == 32
    slab = B // _NUM_WORKERS
    n_grp = slab // _LANES

    mesh = plsc.VectorSubcoreMesh(core_axis_name="c", subcore_axis_name="s")

    @functools.partial(
        pl.kernel,
        mesh=mesh,
        out_type=jax.ShapeDtypeStruct((S, D, B), jnp.float32),
        scratch_types=[
            pltpu.VMEM((S, D), jnp.float32),
            pltpu.VMEM((slab,), jnp.int32),
            pltpu.VMEM((slab,), jnp.int32),
            pltpu.VMEM((slab, 1, D), jnp.float32),
            pltpu.VMEM((slab, 1, D), jnp.float32),
            pltpu.VMEM((D, slab), jnp.float32),
            pltpu.VMEM((D, slab), jnp.float32),
            pltpu.SemaphoreType.DMA,
            pltpu.SemaphoreType.DMA,
            pltpu.SemaphoreType.DMA,
            pltpu.SemaphoreType.DMA,
            pltpu.SemaphoreType.DMA,
            pltpu.SemaphoreType.DMA,
        ],
        compiler_params=pltpu.CompilerParams(use_tc_tiling_on_sc=True,
                                             needs_layout_passes=False),
    )
    def embed(ids_hbm, table_hbm, pos_hbm, out_hbm,
              pos_v, idx0, idx1, rows0, rows1, outT0, outT1,
              isem0, isem1, gsem0, gsem1, osem0, osem1):
        wid = lax.axis_index("s") * _NUM_CORES + lax.axis_index("c")
        b0 = wid * slab
        idx = (idx0, idx1)
        rows = (rows0, rows1)
        outT = (outT0, outT1)
        isem = (isem0, isem1)
        gsem = (gsem0, gsem1)
        osem = (osem0, osem1)

        pltpu.sync_copy(pos_hbm, pos_v)

        def ids_src(s):
            return ids_hbm.at[s, pl.ds(b0, slab)]

        def out_dst(s):
            return out_hbm.at[s, :, pl.ds(b0, slab)]

        pltpu.async_copy(ids_src(0), idx[0], isem[0])
        pltpu.async_copy(ids_src(1), idx[1], isem[1])
        pltpu.make_async_copy(ids_src(0), idx[0], isem[0]).wait()
        pltpu.async_copy(table_hbm.at[idx[0]], rows[0], gsem[0])

        lanes = jnp.arange(_LANES, dtype=jnp.int32)

        def splat(prow, i):
            idxv = jnp.full((_LANES,), i, jnp.int32)
            return lax.gather(
                prow, idxv[:, None],
                lax.GatherDimensionNumbers(
                    offset_dims=(), collapsed_slice_dims=(0,),
                    start_index_map=(0,)),
                (1,), mode=lax.GatherScatterMode.PROMISE_IN_BOUNDS)

        @pl.loop(0, S, step=2)
        def chunk_loop(s0):
            for b in range(2):
                s = s0 + b
                nb = 1 - b

                @pl.when(s + 1 < S)
                def _():
                    pltpu.make_async_copy(ids_src(s + 1), idx[nb],
                                          isem[nb]).wait()

                    @pl.when(s >= 1)
                    def _():
                        pltpu.make_async_copy(outT[nb], out_dst(s - 1),
                                              osem[nb]).wait()

                    pltpu.async_copy(table_hbm.at[idx[nb]], rows[nb],
                                     gsem[nb])

                @pl.when(s + 2 < S)
                def _():
                    pltpu.async_copy(ids_src(s + 2), idx[b], isem[b])

                pltpu.make_async_copy(table_hbm.at[idx[b]], rows[b],
                                      gsem[b]).wait()

                prow = [pos_v[s, pl.ds(h * _LANES, _LANES)]
                        for h in range(D // _LANES)]
                zeros = jnp.zeros((_LANES,), jnp.int32)
                toks = [lanes + (j * _LANES) for j in range(n_grp)]
                for d in range(D):
                    pv = splat(prow[d // _LANES], d % _LANES)
                    col = jnp.full((_LANES,), d, jnp.int32)
                    vals = [plsc.load_gather(rows[b], [toks[j], zeros, col])
                            for j in range(n_grp)]
                    for j in range(n_grp):
                        outT[b][d, pl.ds(j * _LANES, _LANES)] = vals[j] + pv

                pltpu.async_copy(outT[b], out_dst(s), osem[b])

        pltpu.make_async_copy(outT[0], out_dst(S - 2), osem[0]).wait()
        pltpu.make_async_copy(outT[1], out_dst(S - 1), osem[1]).wait()

    return embed


def kernel(input_ids, token_table, pos_table):
    B, S = input_ids.shape
    V, D = token_table.shape
    ids_t = input_ids.astype(jnp.int32).T
    tbl = token_table.reshape(V, 1, D)
    out = _make_kernel(B, S, D, V)(ids_t, tbl, pos_table)
    return out.transpose(2, 0, 1)
